# baseline (speedup 1.0000x reference)
import functools

import jax
import jax.numpy as jnp
from jax import lax
from jax.experimental import pallas as pl
from jax.experimental.pallas import tpu as pltpu
from jax.experimental.pallas import tpu_sc as plsc

F = 26
K = 16
L = 16
NC = 2
NS = 16


@functools.lru_cache(maxsize=None)
def _build(B):
    NW = NC * NS
    RW = B // NW
    CHUNKS = RW // L
    mesh = plsc.VectorSubcoreMesh(core_axis_name="c", subcore_axis_name="s")

    @functools.partial(
        pl.kernel,
        mesh=mesh,
        out_type=jax.ShapeDtypeStruct((B,), jnp.float32),
        compiler_params=pltpu.CompilerParams(needs_layout_passes=False),
        scratch_types=[
            pltpu.VMEM((RW * F,), jnp.float32),
            pltpu.VMEM((F * F * K,), jnp.float32),
            pltpu.SMEM((F * F,), jnp.float32),
            pltpu.VMEM((2 * L,), jnp.float32),
            pltpu.VMEM((L,), jnp.float32),
            pltpu.VMEM((RW,), jnp.float32),
            pltpu.SemaphoreType.DMA,
        ],
    )
    def ffm(x_hbm, emb_hbm, w_hbm, b_hbm, out_hbm,
            x_v, emb_v, m_v, w_v, b_v, out_v, sem):
        cid = lax.axis_index("c")
        sid = lax.axis_index("s")
        wid = sid * NC + cid
        base = wid * RW
        cp = pltpu.async_copy(x_hbm.at[pl.ds(base * F, RW * F)], x_v, sem)
        pltpu.sync_copy(emb_hbm, emb_v)
        pltpu.sync_copy(w_hbm, w_v.at[pl.ds(0, F)])
        pltpu.sync_copy(b_hbm, b_v.at[pl.ds(0, 1)])
        for i in range(F):
            for j in range(i + 1, F):
                a = emb_v[pl.ds((i * F + j) * K, K)]
                c = emb_v[pl.ds((j * F + i) * K, K)]
                m_v[i * F + j] = jnp.sum(a * c)
        cp.wait()
        wa = w_v[pl.ds(0, L)]
        wb = w_v[pl.ds(L, L)]
        ws = [wa[j] for j in range(L)] + [wb[j - L] for j in range(L, F)]
        bias = b_v[pl.ds(0, L)][0]
        colbase = lax.iota(jnp.int32, L) * F

        def chunk(c, carry):
            flat0 = c * (L * F)
            xs = [
                plsc.load_gather(x_v, [colbase + (flat0 + j)])
                for j in range(F)
            ]
            acc = jnp.full((L,), bias, jnp.float32)
            for j in range(F):
                t = jnp.full((L,), ws[j], jnp.float32)
                for i in range(j):
                    t = t + m_v[i * F + j] * xs[i]
                acc = acc + xs[j] * t
            out_v[pl.ds(c * L, L)] = acc
            return carry

        lax.fori_loop(0, CHUNKS, chunk, 0)
        pltpu.sync_copy(out_v, out_hbm.at[pl.ds(base, RW)])

    return ffm


def kernel(x, emb, W, b):
    B = x.shape[0]
    out = _build(B)(x.reshape(-1), emb.reshape(-1), W.reshape(-1), b)
    return out[:, None]


# speedup vs baseline: 1.1761x; 1.1761x over previous
import functools

import jax
import jax.numpy as jnp
from jax import lax
from jax.experimental import pallas as pl
from jax.experimental.pallas import tpu as pltpu
from jax.experimental.pallas import tpu_sc as plsc

F = 26
K = 16
L = 16
NC = 2
NS = 16
SROW = 32


@functools.lru_cache(maxsize=None)
def _build_spair():
    mesh = plsc.VectorSubcoreMesh(core_axis_name="c", subcore_axis_name="s")

    @functools.partial(
        pl.kernel,
        mesh=mesh,
        out_type=jax.ShapeDtypeStruct((SROW * SROW,), jnp.float32),
        compiler_params=pltpu.CompilerParams(needs_layout_passes=False),
        scratch_types=[
            pltpu.VMEM((F * F * K,), jnp.float32),
            pltpu.VMEM((SROW,), jnp.float32),
            pltpu.SemaphoreType.DMA,
        ],
    )
    def spair(emb_hbm, out_hbm, emb_v, out_v, sem):
        cid = lax.axis_index("c")
        sid = lax.axis_index("s")
        w = sid * NC + cid
        i = jnp.minimum(w, F - 1)
        pltpu.sync_copy(emb_hbm, emb_v)
        lane = lax.iota(jnp.int32, L)
        for g in range(SROW // L):
            jv = jnp.minimum(lane + g * L, F - 1)
            acc = jnp.full((L,), 0.0, jnp.float32)
            for k in range(K):
                a = plsc.load_gather(emb_v, [(i * F + jv) * K + k])
                c = plsc.load_gather(emb_v, [(jv * F + i) * K + k])
                acc = acc + a * c
            out_v[pl.ds(g * L, L)] = acc
        pltpu.sync_copy(out_v, out_hbm.at[pl.ds(w * SROW, SROW)])

    return spair


def _tc_body(x_ref, s_ref, w_ref, b_ref, o_ref):
    x = x_ref[...]
    s = s_ref[...][:F, :F]
    ri = lax.broadcasted_iota(jnp.int32, (F, F), 0)
    ci = lax.broadcasted_iota(jnp.int32, (F, F), 1)
    a = jnp.where(ri == ci, 0.0, s) * 0.5
    xa = jnp.dot(x, a, preferred_element_type=jnp.float32)
    t = x * (xa + w_ref[...])
    o_ref[...] = jnp.sum(t, axis=1, keepdims=True) + b_ref[...]


@functools.lru_cache(maxsize=None)
def _build_tc(B, BT):
    return pl.pallas_call(
        _tc_body,
        grid=(B // BT,),
        in_specs=[
            pl.BlockSpec((BT, F), lambda j: (j, 0)),
            pl.BlockSpec((SROW, SROW), lambda j: (0, 0)),
            pl.BlockSpec((1, F), lambda j: (0, 0)),
            pl.BlockSpec((1, 1), lambda j: (0, 0)),
        ],
        out_specs=pl.BlockSpec((BT, 1), lambda j: (j, 0)),
        out_shape=jax.ShapeDtypeStruct((B, 1), jnp.float32),
    )


def kernel(x, emb, W, b):
    B = x.shape[0]
    sflat = _build_spair()(emb.reshape(-1))
    s_pad = sflat.reshape(SROW, SROW)
    bt = min(B, 2048)
    return _build_tc(B, bt)(x, s_pad, W, b.reshape(1, 1))


# speedup vs baseline: 1.7080x; 1.4522x over previous
import functools

import jax
import jax.numpy as jnp
from jax import lax
from jax.experimental import pallas as pl
from jax.experimental.pallas import tpu as pltpu
from jax.experimental.pallas import tpu_sc as plsc

F = 26
K = 16
L = 16
NC = 2
NS = 16
SROW = 32


@functools.lru_cache(maxsize=None)
def _build_spair():
    mesh = plsc.VectorSubcoreMesh(core_axis_name="c", subcore_axis_name="s")

    @functools.partial(
        pl.kernel,
        mesh=mesh,
        out_type=jax.ShapeDtypeStruct((SROW * SROW,), jnp.float32),
        compiler_params=pltpu.CompilerParams(needs_layout_passes=False),
        scratch_types=[
            pltpu.VMEM((F * F * K,), jnp.float32),
            pltpu.VMEM((SROW,), jnp.float32),
            pltpu.SemaphoreType.DMA,
        ],
    )
    def spair(emb_hbm, out_hbm, emb_v, out_v, sem):
        cid = lax.axis_index("c")
        sid = lax.axis_index("s")
        w = sid * NC + cid
        i = jnp.minimum(w, F - 1)
        pltpu.sync_copy(emb_hbm, emb_v)
        lane = lax.iota(jnp.int32, L)
        for g in range(SROW // L):
            jv = jnp.minimum(lane + g * L, F - 1)
            acc = jnp.full((L,), 0.0, jnp.float32)
            for k in range(K):
                a = plsc.load_gather(emb_v, [(i * F + jv) * K + k])
                c = plsc.load_gather(emb_v, [(jv * F + i) * K + k])
                acc = acc + a * c
            out_v[pl.ds(g * L, L)] = acc
        pltpu.sync_copy(out_v, out_hbm.at[pl.ds(w * SROW, SROW)])

    return spair


def _tc_body(x_ref, s_ref, w_ref, b_ref, o_ref):
    x = x_ref[...]
    s = s_ref[...][:F, :F]
    ri = lax.broadcasted_iota(jnp.int32, (F, F), 0)
    ci = lax.broadcasted_iota(jnp.int32, (F, F), 1)
    a = jnp.where(ri == ci, 0.0, s) * 0.5
    xa = jnp.dot(x, a, preferred_element_type=jnp.float32)
    t = x * (xa + w_ref[...])
    o_ref[...] = jnp.sum(t, axis=1, keepdims=True) + b_ref[...]


@functools.lru_cache(maxsize=None)
def _build_tc(B, BT):
    return pl.pallas_call(
        _tc_body,
        grid=(B // BT,),
        in_specs=[
            pl.BlockSpec((BT, F), lambda j: (j, 0)),
            pl.BlockSpec((SROW, SROW), lambda j: (0, 0)),
            pl.BlockSpec((1, F), lambda j: (0, 0)),
            pl.BlockSpec((1, 1), lambda j: (0, 0)),
        ],
        out_specs=pl.BlockSpec((BT, 1), lambda j: (j, 0)),
        out_shape=jax.ShapeDtypeStruct((B, 1), jnp.float32),
    )


def _sp_tc_body(e1_ref, e2_ref, o_ref):
    o_ref[:F, :F] = jnp.sum(e1_ref[...] * e2_ref[...], axis=-1)


@functools.lru_cache(maxsize=None)
def _build_sp_tc():
    return pl.pallas_call(
        _sp_tc_body,
        out_shape=jax.ShapeDtypeStruct((SROW, SROW), jnp.float32),
    )


def kernel(x, emb, W, b):
    B = x.shape[0]
    s_pad = _build_sp_tc()(emb, jnp.swapaxes(emb, 0, 1))
    bt = min(B, 2048)
    return _build_tc(B, bt)(x, s_pad, W, b.reshape(1, 1))


# speedup vs baseline: 1.7921x; 1.0492x over previous
import functools

import jax
import jax.numpy as jnp
from jax import lax
from jax.experimental import pallas as pl
from jax.experimental.pallas import tpu as pltpu
from jax.experimental.pallas import tpu_sc as plsc

F = 26
K = 16
L = 16
NC = 2
NS = 16
SROW = 32


@functools.lru_cache(maxsize=None)
def _build_spair():
    mesh = plsc.VectorSubcoreMesh(core_axis_name="c", subcore_axis_name="s")

    @functools.partial(
        pl.kernel,
        mesh=mesh,
        out_type=jax.ShapeDtypeStruct((SROW * SROW,), jnp.float32),
        compiler_params=pltpu.CompilerParams(needs_layout_passes=False),
        scratch_types=[
            pltpu.VMEM((F * F * K,), jnp.float32),
            pltpu.VMEM((SROW,), jnp.float32),
            pltpu.SemaphoreType.DMA,
        ],
    )
    def spair(emb_hbm, out_hbm, emb_v, out_v, sem):
        cid = lax.axis_index("c")
        sid = lax.axis_index("s")
        w = sid * NC + cid
        i = jnp.minimum(w, F - 1)
        pltpu.sync_copy(emb_hbm, emb_v)
        lane = lax.iota(jnp.int32, L)
        for g in range(SROW // L):
            jv = jnp.minimum(lane + g * L, F - 1)
            acc = jnp.full((L,), 0.0, jnp.float32)
            for k in range(K):
                a = plsc.load_gather(emb_v, [(i * F + jv) * K + k])
                c = plsc.load_gather(emb_v, [(jv * F + i) * K + k])
                acc = acc + a * c
            out_v[pl.ds(g * L, L)] = acc
        pltpu.sync_copy(out_v, out_hbm.at[pl.ds(w * SROW, SROW)])

    return spair


def _tc_body(x_ref, s_ref, w_ref, b_ref, o_ref):
    x = x_ref[...]
    s = s_ref[...][:F, :F]
    ri = lax.broadcasted_iota(jnp.int32, (F, F), 0)
    ci = lax.broadcasted_iota(jnp.int32, (F, F), 1)
    a = jnp.where(ri == ci, 0.0, s) * 0.5
    xa = jnp.dot(x, a, preferred_element_type=jnp.float32)
    t = x * (xa + w_ref[...])
    o_ref[...] = jnp.sum(t, axis=1, keepdims=True) + b_ref[...]


@functools.lru_cache(maxsize=None)
def _build_tc(B, BT):
    return pl.pallas_call(
        _tc_body,
        grid=(B // BT,),
        in_specs=[
            pl.BlockSpec((BT, F), lambda j: (j, 0)),
            pl.BlockSpec((SROW, SROW), lambda j: (0, 0)),
            pl.BlockSpec((1, F), lambda j: (0, 0)),
            pl.BlockSpec((1, 1), lambda j: (0, 0)),
        ],
        out_specs=pl.BlockSpec((BT, 1), lambda j: (j, 0)),
        out_shape=jax.ShapeDtypeStruct((B, 1), jnp.float32),
    )


def _fused_body(x_ref, e1_ref, e2_ref, w_ref, b_ref, o_ref):
    s = jnp.sum(e1_ref[...] * e2_ref[...], axis=-1)
    ri = lax.broadcasted_iota(jnp.int32, (F, F), 0)
    ci = lax.broadcasted_iota(jnp.int32, (F, F), 1)
    a = jnp.where(ri == ci, 0.0, s) * 0.5
    x = x_ref[...]
    xa = jnp.dot(x, a, preferred_element_type=jnp.float32)
    o_ref[...] = jnp.sum(x * (xa + w_ref[...]), axis=1, keepdims=True) + b_ref[...]


@functools.lru_cache(maxsize=None)
def _build_fused(B, BT):
    return pl.pallas_call(
        _fused_body,
        grid=(B // BT,),
        in_specs=[
            pl.BlockSpec((BT, F), lambda j: (j, 0)),
            pl.BlockSpec((F, F, K), lambda j: (0, 0, 0)),
            pl.BlockSpec((F, F, K), lambda j: (0, 0, 0)),
            pl.BlockSpec((1, F), lambda j: (0, 0)),
            pl.BlockSpec((1, 1), lambda j: (0, 0)),
        ],
        out_specs=pl.BlockSpec((BT, 1), lambda j: (j, 0)),
        out_shape=jax.ShapeDtypeStruct((B, 1), jnp.float32),
    )


def kernel(x, emb, W, b):
    B = x.shape[0]
    bt = min(B, 2048)
    return _build_fused(B, bt)(x, emb, jnp.swapaxes(emb, 0, 1), W,
                               b.reshape(1, 1))


# speedup vs baseline: 1.9381x; 1.0815x over previous
import functools

import jax
import jax.numpy as jnp
from jax import lax
from jax.experimental import pallas as pl
from jax.experimental.pallas import tpu as pltpu
from jax.experimental.pallas import tpu_sc as plsc

F = 26
K = 16
L = 16
NC = 2
NS = 16
SROW = 32


@functools.lru_cache(maxsize=None)
def _build_spair():
    mesh = plsc.VectorSubcoreMesh(core_axis_name="c", subcore_axis_name="s")

    @functools.partial(
        pl.kernel,
        mesh=mesh,
        out_type=jax.ShapeDtypeStruct((SROW * SROW,), jnp.float32),
        compiler_params=pltpu.CompilerParams(needs_layout_passes=False),
        scratch_types=[
            pltpu.VMEM((F * F * K,), jnp.float32),
            pltpu.VMEM((SROW,), jnp.float32),
            pltpu.SemaphoreType.DMA,
        ],
    )
    def spair(emb_hbm, out_hbm, emb_v, out_v, sem):
        cid = lax.axis_index("c")
        sid = lax.axis_index("s")
        w = sid * NC + cid
        i = jnp.minimum(w, F - 1)
        pltpu.sync_copy(emb_hbm, emb_v)
        lane = lax.iota(jnp.int32, L)
        for g in range(SROW // L):
            jv = jnp.minimum(lane + g * L, F - 1)
            acc = jnp.full((L,), 0.0, jnp.float32)
            for k in range(K):
                a = plsc.load_gather(emb_v, [(i * F + jv) * K + k])
                c = plsc.load_gather(emb_v, [(jv * F + i) * K + k])
                acc = acc + a * c
            out_v[pl.ds(g * L, L)] = acc
        pltpu.sync_copy(out_v, out_hbm.at[pl.ds(w * SROW, SROW)])

    return spair


def _tc_body(x_ref, s_ref, w_ref, b_ref, o_ref):
    x = x_ref[...]
    s = s_ref[...][:F, :F]
    ri = lax.broadcasted_iota(jnp.int32, (F, F), 0)
    ci = lax.broadcasted_iota(jnp.int32, (F, F), 1)
    a = jnp.where(ri == ci, 0.0, s) * 0.5
    xa = jnp.dot(x, a, preferred_element_type=jnp.float32)
    t = x * (xa + w_ref[...])
    o_ref[...] = jnp.sum(t, axis=1, keepdims=True) + b_ref[...]


@functools.lru_cache(maxsize=None)
def _build_tc(B, BT):
    return pl.pallas_call(
        _tc_body,
        grid=(B // BT,),
        in_specs=[
            pl.BlockSpec((BT, F), lambda j: (j, 0)),
            pl.BlockSpec((SROW, SROW), lambda j: (0, 0)),
            pl.BlockSpec((1, F), lambda j: (0, 0)),
            pl.BlockSpec((1, 1), lambda j: (0, 0)),
        ],
        out_specs=pl.BlockSpec((BT, 1), lambda j: (j, 0)),
        out_shape=jax.ShapeDtypeStruct((B, 1), jnp.float32),
    )


def _fused_body(x_ref, e1_ref, e2_ref, w_ref, b_ref, o_ref):
    s = jnp.sum(e1_ref[...] * e2_ref[...], axis=-1)
    ri = lax.broadcasted_iota(jnp.int32, (F, F), 0)
    ci = lax.broadcasted_iota(jnp.int32, (F, F), 1)
    a = jnp.where(ri == ci, 0.0, s) * 0.5
    x = x_ref[...]
    xa = jnp.dot(x, a, preferred_element_type=jnp.float32)
    o_ref[...] = jnp.sum(x * (xa + w_ref[...]), axis=1, keepdims=True) + b_ref[...]


@functools.lru_cache(maxsize=None)
def _build_fused(B, BT):
    return pl.pallas_call(
        _fused_body,
        grid=(B // BT,),
        in_specs=[
            pl.BlockSpec((BT, F), lambda j: (j, 0)),
            pl.BlockSpec((F, F, K), lambda j: (0, 0, 0)),
            pl.BlockSpec((F, F, K), lambda j: (0, 0, 0)),
            pl.BlockSpec((1, F), lambda j: (0, 0)),
            pl.BlockSpec((1, 1), lambda j: (0, 0)),
        ],
        out_specs=pl.BlockSpec((BT, 1), lambda j: (j, 0)),
        out_shape=jax.ShapeDtypeStruct((B, 1), jnp.float32),
    )


def kernel(x, emb, W, b):
    B = x.shape[0]
    bt = min(B, 16384)
    return _build_fused(B, bt)(x, emb, jnp.swapaxes(emb, 0, 1), W,
                               b.reshape(1, 1))


# speedup vs baseline: 6.1777x; 3.1875x over previous
import functools

import jax
import jax.numpy as jnp
from jax import lax
from jax.experimental import pallas as pl
from jax.experimental.pallas import tpu as pltpu

F = 26
K = 16


def _body(xt_ref, ev_ref, w_ref, b_ref, o_ref, a_scr):
    @pl.when(pl.program_id(0) == 0)
    def _():
        ev = ev_ref[...]
        s = jnp.zeros((F, F), jnp.float32)
        for k in range(K):
            sk = ev[:, k, :]
            s = s + sk * sk.T
        ri = lax.broadcasted_iota(jnp.int32, (F, F), 0)
        ci = lax.broadcasted_iota(jnp.int32, (F, F), 1)
        a_scr[...] = jnp.where(ri == ci, 0.0, s) * 0.5

    xt = xt_ref[...]
    ax = jnp.dot(a_scr[...], xt, preferred_element_type=jnp.float32)
    t = xt * (ax + w_ref[...])
    o_ref[...] = jnp.sum(t, axis=0) + b_ref[0, 0]


@functools.lru_cache(maxsize=None)
def _build(B, BT):
    return pl.pallas_call(
        _body,
        grid=(B // BT,),
        in_specs=[
            pl.BlockSpec((F, BT), lambda j: (0, j)),
            pl.BlockSpec((F, K, F), lambda j: (0, 0, 0)),
            pl.BlockSpec((F, 1), lambda j: (0, 0)),
            pl.BlockSpec((1, 1), lambda j: (0, 0)),
        ],
        out_specs=pl.BlockSpec((BT,), lambda j: (j,)),
        out_shape=jax.ShapeDtypeStruct((B,), jnp.float32),
        scratch_shapes=[pltpu.VMEM((F, F), jnp.float32)],
    )


def kernel(x, emb, W, b):
    B = x.shape[0]
    bt = min(B, 2048)
    out = _build(B, bt)(x.T, emb.transpose(0, 2, 1), W.T, b.reshape(1, 1))
    return out[:, None]


# speedup vs baseline: 7.3403x; 1.1882x over previous
import functools

import jax
import jax.numpy as jnp
from jax import lax
from jax.experimental import pallas as pl
from jax.experimental.pallas import tpu as pltpu

F = 26
K = 16


def _body(xt_ref, ev_ref, w_ref, b_ref, o_ref, a_scr):
    @pl.when(pl.program_id(0) == 0)
    def _():
        ev = ev_ref[...]
        s = jnp.zeros((F, F), jnp.float32)
        for k in range(K):
            sk = ev[:, k, :]
            s = s + sk * sk.T
        ri = lax.broadcasted_iota(jnp.int32, (F, F), 0)
        ci = lax.broadcasted_iota(jnp.int32, (F, F), 1)
        a_scr[...] = jnp.where(ri == ci, 0.0, s) * 0.5

    xt = xt_ref[...]
    ax = jnp.dot(a_scr[...], xt, preferred_element_type=jnp.float32)
    xw = jnp.dot(w_ref[...], xt, preferred_element_type=jnp.float32)
    o_ref[...] = jnp.sum(xt * ax, axis=0) + xw[0] + b_ref[0, 0]


@functools.lru_cache(maxsize=None)
def _build(B, BT):
    return pl.pallas_call(
        _body,
        grid=(B // BT,),
        in_specs=[
            pl.BlockSpec((F, BT), lambda j: (0, j)),
            pl.BlockSpec((F, K, F), lambda j: (0, 0, 0)),
            pl.BlockSpec((1, F), lambda j: (0, 0)),
            pl.BlockSpec((1, 1), lambda j: (0, 0)),
        ],
        out_specs=pl.BlockSpec((BT,), lambda j: (j,)),
        out_shape=jax.ShapeDtypeStruct((B,), jnp.float32),
        scratch_shapes=[pltpu.VMEM((F, F), jnp.float32)],
    )


def kernel(x, emb, W, b):
    B = x.shape[0]
    bt = min(B, 2048)
    out = _build(B, bt)(x.T, emb.transpose(0, 2, 1), W, b.reshape(1, 1))
    return out[:, None]
